# baseline (speedup 1.0000x reference)
import functools

import jax
import jax.numpy as jnp
from jax import lax
from jax.experimental import pallas as pl
from jax.experimental.pallas import tpu as pltpu
from jax.experimental.pallas import tpu_sc as plsc

_N = 10000
_E = 320000
_H = 128
_NSC = 2
_NS = 16
_E2 = _E // 2
_IR = 100
_BC = 200
_RPC = _BC // _IR
_IRS = 80
_BCS = 80
_RPCS = _BCS // _IRS
_EPT = _E2 // _NS
_NCH = _EPT // _BCS
_EPTG = _E2 // (_NSC * _NS)
_NCHG = _EPTG // _BC
_AR0 = 624
_ARN = 640
_HF = _H // 2

_MESH = plsc.VectorSubcoreMesh(
    core_axis_name="c", subcore_axis_name="s", num_cores=_NSC, num_subcores=_NS)


@functools.partial(
    pl.kernel,
    out_type=jax.ShapeDtypeStruct((_E2, _H), jnp.float32),
    mesh=_MESH,
    scratch_types=[
        pltpu.VMEM((_BC, _H), jnp.float32),
        pltpu.VMEM((_BC, _H), jnp.float32),
        pltpu.VMEM((_RPC, _IR), jnp.int32),
        pltpu.VMEM((_RPC, _IR), jnp.int32),
        pltpu.SemaphoreType.DMA,
    ],
)
def _sc_gather(dh, eh, src_i, dst_i, g, buf_d, buf_e, src_b, dst_b, sem):
    c = lax.axis_index("c")
    s = lax.axis_index("s")

    def chunk(j, carry):
        base = (c * _NS + s) * _EPTG + j * _BC
        pltpu.sync_copy(src_i.at[c, s, j], src_b)
        pltpu.sync_copy(dst_i.at[c, s, j], dst_b)
        descs = []
        for q in range(_RPC):
            descs.append(pltpu.async_copy(
                dh.at[src_b.at[q]], buf_d.at[pl.ds(q * _IR, _IR)], sem))
            descs.append(pltpu.async_copy(
                eh.at[dst_b.at[q]], buf_e.at[pl.ds(q * _IR, _IR)], sem))
        for d in descs:
            d.wait()

        def addrow(r, carry2):
            for v in range(_H // 16):
                sl = pl.ds(v * 16, 16)
                buf_d[r, sl] = buf_d[r, sl] + buf_e[r, sl]
            return carry2

        lax.fori_loop(0, _BC, addrow, 0)
        pltpu.sync_copy(buf_d, g.at[pl.ds(base, _BC)])
        return carry

    lax.fori_loop(0, _NCHG, chunk, 0)


_SC_SCRATCH = [
    pltpu.VMEM((_BCS, _H), jnp.float32),
    pltpu.VMEM((_BCS, _H), jnp.float32),
    pltpu.VMEM((_RPCS, _IRS), jnp.int32),
    pltpu.VMEM((_RPCS, _IRS), jnp.int32),
    pltpu.VMEM_SHARED((_N, _H), jnp.float32),
    pltpu.SemaphoreType.DMA,
    pltpu.SemaphoreType.DMA,
]


def _scatter_body(seeded, sig, bh, src_i, dst_i, acc_in, acc_out,
                  sig_b, b_b, src_b, dst_b, acc, gsem, ssem):
    c = lax.axis_index("c")
    s = lax.axis_index("s")

    r0 = s * _AR0
    if seeded:
        pltpu.sync_copy(acc_in.at[c, pl.ds(r0, _ARN)], acc.at[pl.ds(r0, _ARN)])
    else:
        zeros = jnp.zeros((16,), jnp.float32)

        def zrow(r, carry):
            for v in range(_H // 16):
                b_b[r, pl.ds(v * 16, 16)] = zeros
            return carry

        lax.fori_loop(0, _BCS, zrow, 0)
        for t in range(_ARN // _BCS):
            pltpu.sync_copy(b_b, acc.at[pl.ds(r0 + t * _BCS, _BCS)])
        rem = _ARN % _BCS
        if rem:
            pltpu.sync_copy(b_b.at[pl.ds(0, rem)],
                            acc.at[pl.ds(r0 + _ARN - rem, rem)])
    plsc.subcore_barrier()

    def chunk(j, carry):
        base = s * _EPT + j * _BCS
        pltpu.sync_copy(dst_i.at[s, j], dst_b)
        pltpu.sync_copy(sig.at[pl.ds(base, _BCS)], sig_b)

        @pl.when(c == 0)
        def _num():
            pltpu.sync_copy(src_i.at[s, j], src_b)
            descs = []
            for q in range(_RPCS):
                descs.append(pltpu.async_copy(
                    bh.at[src_b.at[q]], b_b.at[pl.ds(q * _IRS, _IRS)], gsem))
            for d in descs:
                d.wait()

            def mulrow(r, carry2):
                for v in range(_H // 16):
                    sl = pl.ds(v * 16, 16)
                    b_b[r, sl] = sig_b[r, sl] * b_b[r, sl]
                return carry2

            lax.fori_loop(0, _BCS, mulrow, 0)
            sdescs = []
            for q in range(_RPCS):
                sdescs.append(pltpu.async_copy(
                    b_b.at[pl.ds(q * _IRS, _IRS)], acc.at[dst_b.at[q]], ssem,
                    add=True))
            for d in sdescs:
                d.wait()

        @pl.when(c == 1)
        def _den():
            sdescs = []
            for q in range(_RPCS):
                sdescs.append(pltpu.async_copy(
                    sig_b.at[pl.ds(q * _IRS, _IRS)], acc.at[dst_b.at[q]], ssem,
                    add=True))
            for d in sdescs:
                d.wait()

        return carry

    lax.fori_loop(0, _NCH, chunk, 0)
    plsc.subcore_barrier()
    pltpu.sync_copy(acc.at[pl.ds(r0, _ARN)], acc_out.at[c, pl.ds(r0, _ARN)])


@functools.partial(
    pl.kernel,
    out_type=jax.ShapeDtypeStruct((_NSC, _N, _H), jnp.float32),
    mesh=_MESH,
    scratch_types=_SC_SCRATCH,
)
def _sc_scatter0(sig, bh, src_i, dst_i, acc_out, *scratch):
    _scatter_body(False, sig, bh, src_i, dst_i, None, acc_out, *scratch)


@functools.partial(
    pl.kernel,
    out_type=jax.ShapeDtypeStruct((_NSC, _N, _H), jnp.float32),
    mesh=_MESH,
    scratch_types=_SC_SCRATCH,
)
def _sc_scatter1(sig, bh, src_i, dst_i, acc_in, acc_out, *scratch):
    _scatter_body(True, sig, bh, src_i, dst_i, acc_in, acc_out, *scratch)


_BN = 2000
_BE = 2000


def _node_matmuls(hh, w_ref, b_ref, hh_ref, ah_ref, bh_ref, dh_ref, eh_ref):
    hh_ref[...] = hh
    ah_ref[...] = jnp.dot(hh, w_ref[0], preferred_element_type=jnp.float32) + b_ref[0]
    bh_ref[...] = jnp.dot(hh, w_ref[1], preferred_element_type=jnp.float32) + b_ref[1]
    dh_ref[...] = jnp.dot(hh, w_ref[2], preferred_element_type=jnp.float32) + b_ref[2]
    eh_ref[...] = jnp.dot(hh, w_ref[3], preferred_element_type=jnp.float32) + b_ref[3]


def _node0_body(h_ref, emb_ref, w_ref, b_ref, *out_refs):
    onehot = (h_ref[...] == lax.broadcasted_iota(jnp.int32, (_BN, _H), 1)
              ).astype(jnp.float32)
    hh = jnp.dot(onehot, emb_ref[...], preferred_element_type=jnp.float32)
    _node_matmuls(hh, w_ref, b_ref, *out_refs)


def _hupdate(hprev_ref, ah_ref, acc_ref):
    return hprev_ref[...] + jnp.maximum(
        ah_ref[...] + acc_ref[0] / (acc_ref[1] + 1e-6), 0.0)


def _node_body(hprev_ref, ahprev_ref, acc_ref, w_ref, b_ref, *out_refs):
    hh = _hupdate(hprev_ref, ahprev_ref, acc_ref)
    _node_matmuls(hh, w_ref, b_ref, *out_refs)


def _node_specs():
    ins = [
        pl.BlockSpec((5, _H, _H), lambda i: (0, 0, 0)),
        pl.BlockSpec((5, 1, _H), lambda i: (0, 0, 0)),
    ]
    outs = [pl.BlockSpec((_BN, _H), lambda i: (i, 0))] * 5
    out_shape = [jax.ShapeDtypeStruct((_N, _H), jnp.float32)] * 5
    return ins, outs, out_shape


def _tc_node0(h2, emb, w, b):
    ins, outs, out_shape = _node_specs()
    return pl.pallas_call(
        _node0_body,
        grid=(_N // _BN,),
        in_specs=[pl.BlockSpec((_BN, 1), lambda i: (i, 0)),
                  pl.BlockSpec((_H, _H), lambda i: (0, 0))] + ins,
        out_specs=outs,
        out_shape=out_shape,
    )(h2, emb, w, b)


def _tc_node(hprev, ahprev, acc, w, b):
    ins, outs, out_shape = _node_specs()
    return pl.pallas_call(
        _node_body,
        grid=(_N // _BN,),
        in_specs=[pl.BlockSpec((_BN, _H), lambda i: (i, 0)),
                  pl.BlockSpec((_BN, _H), lambda i: (i, 0)),
                  pl.BlockSpec((2, _BN, _H), lambda i: (0, i, 0))] + ins,
        out_specs=outs,
        out_shape=out_shape,
    )(hprev, ahprev, acc, w, b)


def _edge_math(ee, g_ref, w_ref, b_ref, sig_ref, eenew_ref):
    x = jnp.dot(ee, w_ref[...], preferred_element_type=jnp.float32)
    x = x + b_ref[...] + g_ref[...]
    sig_ref[...] = jax.nn.sigmoid(x)
    if eenew_ref is not None:
        eenew_ref[...] = ee + jnp.maximum(x, 0.0)


def _edge0_body(e_ref, we_ref, be_ref, g_ref, w_ref, b_ref, sig_ref, eenew_ref):
    ee = e_ref[...] * we_ref[...] + be_ref[...]
    _edge_math(ee, g_ref, w_ref, b_ref, sig_ref, eenew_ref)


def _edge_body(ee_ref, g_ref, w_ref, b_ref, sig_ref, eenew_ref=None):
    _edge_math(ee_ref[...], g_ref, w_ref, b_ref, sig_ref, eenew_ref)


_EDGE_SPEC = pl.BlockSpec((_BE, _H), lambda i: (i, 0))
_EDGE_SHAPE = jax.ShapeDtypeStruct((_E2, _H), jnp.float32)


def _edge_wspecs():
    return [_EDGE_SPEC,
            pl.BlockSpec((_H, _H), lambda i: (0, 0)),
            pl.BlockSpec((1, _H), lambda i: (0, 0))]


def _tc_edge0(e, we, be, g, w4, b4, off):
    return pl.pallas_call(
        _edge0_body,
        grid=(_E2 // _BE,),
        in_specs=[pl.BlockSpec((_BE, 1), lambda i: (i + off, 0)),
                  pl.BlockSpec((1, _H), lambda i: (0, 0)),
                  pl.BlockSpec((1, _H), lambda i: (0, 0))] + _edge_wspecs(),
        out_specs=[_EDGE_SPEC, _EDGE_SPEC],
        out_shape=[_EDGE_SHAPE, _EDGE_SHAPE],
    )(e, we, be, g, w4, b4)


def _tc_edge(ee, g, w4, b4, want_ee):
    out_specs = [_EDGE_SPEC, _EDGE_SPEC] if want_ee else [_EDGE_SPEC]
    out_shape = [_EDGE_SHAPE, _EDGE_SHAPE] if want_ee else [_EDGE_SHAPE]
    return pl.pallas_call(
        _edge_body,
        grid=(_E2 // _BE,),
        in_specs=[_EDGE_SPEC] + _edge_wspecs(),
        out_specs=out_specs,
        out_shape=out_shape,
    )(ee, g, w4, b4)


def _readout_body(hprev_ref, ahprev_ref, acc_ref, w0, b0, w1, b1, w2, b2, y_ref):
    hh = _hupdate(hprev_ref, ahprev_ref, acc_ref)
    y = jnp.maximum(jnp.dot(hh, w0[...], preferred_element_type=jnp.float32)
                    + b0[...], 0.0)
    y = jnp.maximum(jnp.dot(y, w1[...], preferred_element_type=jnp.float32)
                    + b1[...], 0.0)
    y_ref[...] = jnp.dot(y, w2[...], preferred_element_type=jnp.float32) + b2[...]


def _tc_readout(hprev, ahprev, acc, W0, b0, W1, b1, W2, b2):
    H2, H4, NC = W0.shape[1], W1.shape[1], W2.shape[1]
    return pl.pallas_call(
        _readout_body,
        grid=(_N // _BN,),
        in_specs=[pl.BlockSpec((_BN, _H), lambda i: (i, 0)),
                  pl.BlockSpec((_BN, _H), lambda i: (i, 0)),
                  pl.BlockSpec((2, _BN, _H), lambda i: (0, i, 0)),
                  pl.BlockSpec((_H, H2), lambda i: (0, 0)),
                  pl.BlockSpec((1, H2), lambda i: (0, 0)),
                  pl.BlockSpec((H2, H4), lambda i: (0, 0)),
                  pl.BlockSpec((1, H4), lambda i: (0, 0)),
                  pl.BlockSpec((H4, NC), lambda i: (0, 0)),
                  pl.BlockSpec((1, NC), lambda i: (0, 0))],
        out_specs=pl.BlockSpec((_BN, NC), lambda i: (i, 0)),
        out_shape=jax.ShapeDtypeStruct((_N, NC), jnp.float32),
    )(hprev, ahprev, acc, W0, b0, W1, b1, W2, b2)


def kernel(h, edge_index, e, emb_h, We, be, layerW, layerB, W0, b0, W1, b1, W2, b2):
    L = layerW.shape[0]
    src = edge_index[0].astype(jnp.int32)
    dst = edge_index[1].astype(jnp.int32)
    srcH = src.reshape(2, _E2)
    dstH = dst.reshape(2, _E2)
    src_g = [srcH[p].reshape(_NSC, _NS, _NCHG, _RPC, _IR) for p in range(2)]
    dst_g = [dstH[p].reshape(_NSC, _NS, _NCHG, _RPC, _IR) for p in range(2)]
    src_s = [srcH[p].reshape(_NS, _NCH, _RPCS, _IRS) for p in range(2)]
    dst_s = [dstH[p].reshape(_NS, _NCH, _RPCS, _IRS) for p in range(2)]

    h2 = h.astype(jnp.int32).reshape(_N, 1)
    lW = layerW.astype(jnp.float32)
    lB = layerB.reshape(L, 5, 1, _H).astype(jnp.float32)

    hh = ah = acc = None
    eeh = [None, None]
    sigh = [None, None]
    for l in range(L):
        if l == 0:
            hh, ah, bh, dh, eh = _tc_node0(h2, emb_h, lW[0], lB[0])
        else:
            hh, ah, bh, dh, eh = _tc_node(hh, ah, acc, lW[l], lB[l])
        gh = [_sc_gather(dh, eh, src_g[p], dst_g[p]) for p in range(2)]
        w4 = lW[l, 4]
        b4 = lB[l, 4]
        for p in range(2):
            if l == 0:
                sigh[p], eeh[p] = _tc_edge0(
                    e, We.reshape(1, _H), be.reshape(1, _H),
                    gh[p], w4, b4, p * (_E2 // _BE))
            elif l < L - 1:
                sigh[p], eeh[p] = _tc_edge(eeh[p], gh[p], w4, b4, True)
            else:
                (sigh[p],) = _tc_edge(eeh[p], gh[p], w4, b4, False)
        accA = _sc_scatter0(sigh[0], bh, src_s[0], dst_s[0])
        acc = _sc_scatter1(sigh[1], bh, src_s[1], dst_s[1], accA)

    return _tc_readout(hh, ah, acc, W0, b0.reshape(1, -1), W1, b1.reshape(1, -1),
                       W2, b2.reshape(1, -1))


# speedup vs baseline: 1.0504x; 1.0504x over previous
import functools

import jax
import jax.numpy as jnp
from jax import lax
from jax.experimental import pallas as pl
from jax.experimental.pallas import tpu as pltpu
from jax.experimental.pallas import tpu_sc as plsc

_N = 10000
_E = 320000
_H = 128
_NSC = 2
_NS = 16
_E2 = _E // 2
_IR = 100
_BC = 200
_RPC = _BC // _IR
_IRS = 100
_BCS = 200
_RPCS = _BCS // _IRS
_EPTH = _E2 // _NS
_NCHH = _EPTH // _BCS
_EPTG = _E2 // (_NSC * _NS)
_NCHG = _EPTG // _BC
_HC = _H // 2
_AR0 = 624
_ARN = 640

_MESH = plsc.VectorSubcoreMesh(
    core_axis_name="c", subcore_axis_name="s", num_cores=_NSC, num_subcores=_NS)


@functools.partial(
    pl.kernel,
    out_type=jax.ShapeDtypeStruct((_E2, _H), jnp.float32),
    mesh=_MESH,
    scratch_types=[
        pltpu.VMEM((_BC, _H), jnp.float32),
        pltpu.VMEM((_BC, _H), jnp.float32),
        pltpu.VMEM((_RPC, _IR), jnp.int32),
        pltpu.VMEM((_RPC, _IR), jnp.int32),
        pltpu.SemaphoreType.DMA,
    ],
)
def _sc_gather(dh, eh, src_i, dst_i, g, buf_d, buf_e, src_b, dst_b, sem):
    c = lax.axis_index("c")
    s = lax.axis_index("s")

    def chunk(j, carry):
        base = (c * _NS + s) * _EPTG + j * _BC
        pltpu.sync_copy(src_i.at[c, s, j], src_b)
        pltpu.sync_copy(dst_i.at[c, s, j], dst_b)
        descs = []
        for q in range(_RPC):
            descs.append(pltpu.async_copy(
                dh.at[src_b.at[q]], buf_d.at[pl.ds(q * _IR, _IR)], sem))
            descs.append(pltpu.async_copy(
                eh.at[dst_b.at[q]], buf_e.at[pl.ds(q * _IR, _IR)], sem))
        for d in descs:
            d.wait()

        def addrow(r, carry2):
            for v in range(_H // 16):
                sl = pl.ds(v * 16, 16)
                buf_d[r, sl] = buf_d[r, sl] + buf_e[r, sl]
            return carry2

        lax.fori_loop(0, _BC, addrow, 0)
        pltpu.sync_copy(buf_d, g.at[pl.ds(base, _BC)])
        return carry

    lax.fori_loop(0, _NCHG, chunk, 0)


_SGS = 40

_SC_SCRATCH = [
    pltpu.VMEM((_SGS, _HC), jnp.float32),
    pltpu.VMEM((_BCS, _H), jnp.float32),
    pltpu.VMEM((_RPCS, _IRS), jnp.int32),
    pltpu.VMEM((_RPCS, _IRS), jnp.int32),
    pltpu.VMEM_SHARED((_N, _H), jnp.float32),
    pltpu.SemaphoreType.DMA,
    pltpu.SemaphoreType.DMA,
]


@functools.partial(
    pl.kernel,
    out_type=jax.ShapeDtypeStruct((_NSC, _N, _H), jnp.float32),
    mesh=_MESH,
    scratch_types=_SC_SCRATCH,
)
def _sc_scatter(sig0, sig1, bh, src0, dst0, src1, dst1,
                acc_out, sig_b, b_b, src_b, dst_b, acc, gsem, ssem):
    c = lax.axis_index("c")
    s = lax.axis_index("s")

    r0 = s * _AR0
    zeros = jnp.zeros((16,), jnp.float32)

    def zrow(r, carry):
        for v in range(_H // 16):
            b_b[r, pl.ds(v * 16, 16)] = zeros
        return carry

    lax.fori_loop(0, _BCS, zrow, 0)
    for t in range(_ARN // _BCS):
        pltpu.sync_copy(b_b, acc.at[pl.ds(r0 + t * _BCS, _BCS)])
    rem = _ARN % _BCS
    if rem:
        pltpu.sync_copy(b_b.at[pl.ds(0, rem)],
                        acc.at[pl.ds(r0 + _ARN - rem, rem)])
    plsc.subcore_barrier()

    def make_chunk(sig_p, src_p, dst_p):
        def chunk(j, carry):
            base = s * _EPTH + j * _BCS
            pltpu.sync_copy(src_p.at[s, j], src_b)
            pltpu.sync_copy(dst_p.at[s, j], dst_b)
            descs = []
            for q in range(_RPCS):
                descs.append(pltpu.async_copy(
                    bh.at[src_b.at[q]], b_b.at[pl.ds(q * _IRS, _IRS)],
                    gsem))
            for d in descs:
                d.wait()

            for q in range(_BCS // _SGS):
                off = q * _SGS
                pltpu.sync_copy(sig_p.at[c, pl.ds(base + off, _SGS)], sig_b)

                def mulrow_lo(r, carry2, off=off):
                    for v in range(_HC // 16):
                        sl = pl.ds(v * 16, 16)
                        sv = sig_b[r, sl]
                        b_b[off + r, sl] = sv * b_b[off + r, sl]
                        b_b[off + r, pl.ds(_HC + v * 16, 16)] = sv
                    return carry2

                def mulrow_hi(r, carry2, off=off):
                    for v in range(_HC // 16):
                        sl = pl.ds(v * 16, 16)
                        sv = sig_b[r, sl]
                        b_b[off + r, sl] = sv * b_b[off + r, pl.ds(_HC + v * 16, 16)]
                        b_b[off + r, pl.ds(_HC + v * 16, 16)] = sv
                    return carry2

                @pl.when(c == 0)
                def _lo(mulrow_lo=mulrow_lo):
                    lax.fori_loop(0, _SGS, mulrow_lo, 0)

                @pl.when(c == 1)
                def _hi(mulrow_hi=mulrow_hi):
                    lax.fori_loop(0, _SGS, mulrow_hi, 0)

            sdescs = []
            for q in range(_RPCS):
                sdescs.append(pltpu.async_copy(
                    b_b.at[pl.ds(q * _IRS, _IRS)], acc.at[dst_b.at[q]], ssem,
                    add=True))
            for d in sdescs:
                d.wait()
            return carry
        return chunk

    lax.fori_loop(0, _NCHH, make_chunk(sig0, src0, dst0), 0)
    lax.fori_loop(0, _NCHH, make_chunk(sig1, src1, dst1), 0)
    plsc.subcore_barrier()
    pltpu.sync_copy(acc.at[pl.ds(r0, _ARN)], acc_out.at[c, pl.ds(r0, _ARN)])


_BN = 2000
_BE = 2000


def _node_matmuls(hh, w_ref, b_ref, hh_ref, ah_ref, bh_ref, dh_ref, eh_ref):
    hh_ref[...] = hh
    ah_ref[...] = jnp.dot(hh, w_ref[0], preferred_element_type=jnp.float32) + b_ref[0]
    bh_ref[...] = jnp.dot(hh, w_ref[1], preferred_element_type=jnp.float32) + b_ref[1]
    dh_ref[...] = jnp.dot(hh, w_ref[2], preferred_element_type=jnp.float32) + b_ref[2]
    eh_ref[...] = jnp.dot(hh, w_ref[3], preferred_element_type=jnp.float32) + b_ref[3]


def _node0_body(h_ref, emb_ref, w_ref, b_ref, *out_refs):
    onehot = (h_ref[...] == lax.broadcasted_iota(jnp.int32, (_BN, _H), 1)
              ).astype(jnp.float32)
    hh = jnp.dot(onehot, emb_ref[...], preferred_element_type=jnp.float32)
    _node_matmuls(hh, w_ref, b_ref, *out_refs)


def _hupdate(hprev_ref, ah_ref, acc_ref):
    num = jnp.concatenate([acc_ref[0, :, :_HC], acc_ref[1, :, :_HC]], axis=1)
    den = jnp.concatenate([acc_ref[0, :, _HC:], acc_ref[1, :, _HC:]], axis=1)
    return hprev_ref[...] + jnp.maximum(
        ah_ref[...] + num / (den + 1e-6), 0.0)


def _node_body(hprev_ref, ahprev_ref, acc_ref, w_ref, b_ref, *out_refs):
    hh = _hupdate(hprev_ref, ahprev_ref, acc_ref)
    _node_matmuls(hh, w_ref, b_ref, *out_refs)


_ACC_SPECS = [pl.BlockSpec((_NSC, _BN, _H), lambda i: (0, i, 0))]


def _node_specs():
    ins = [
        pl.BlockSpec((5, _H, _H), lambda i: (0, 0, 0)),
        pl.BlockSpec((5, 1, _H), lambda i: (0, 0, 0)),
    ]
    outs = [pl.BlockSpec((_BN, _H), lambda i: (i, 0))] * 5
    out_shape = [jax.ShapeDtypeStruct((_N, _H), jnp.float32)] * 5
    return ins, outs, out_shape


def _tc_node0(h2, emb, w, b):
    ins, outs, out_shape = _node_specs()
    return pl.pallas_call(
        _node0_body,
        grid=(_N // _BN,),
        in_specs=[pl.BlockSpec((_BN, 1), lambda i: (i, 0)),
                  pl.BlockSpec((_H, _H), lambda i: (0, 0))] + ins,
        out_specs=outs,
        out_shape=out_shape,
    )(h2, emb, w, b)


def _tc_node(hprev, ahprev, acc, w, b):
    ins, outs, out_shape = _node_specs()
    return pl.pallas_call(
        _node_body,
        grid=(_N // _BN,),
        in_specs=[pl.BlockSpec((_BN, _H), lambda i: (i, 0)),
                  pl.BlockSpec((_BN, _H), lambda i: (i, 0))] + _ACC_SPECS + ins,
        out_specs=outs,
        out_shape=out_shape,
    )(hprev, ahprev, acc, w, b)


def _edge_math(ee, g_ref, w_ref, b_ref, sig_ref, eenew_ref):
    x = jnp.dot(ee, w_ref[...], preferred_element_type=jnp.float32)
    x = x + b_ref[...] + g_ref[...]
    sig = jax.nn.sigmoid(x)
    sig_ref[0] = sig[:, :_HC]
    sig_ref[1] = sig[:, _HC:]
    if eenew_ref is not None:
        eenew_ref[...] = ee + jnp.maximum(x, 0.0)


def _edge0_body(e_ref, we_ref, be_ref, g_ref, w_ref, b_ref, sig_ref, eenew_ref):
    ee = e_ref[...] * we_ref[...] + be_ref[...]
    _edge_math(ee, g_ref, w_ref, b_ref, sig_ref, eenew_ref)


def _edge_body(ee_ref, g_ref, w_ref, b_ref, sig_ref, eenew_ref=None):
    _edge_math(ee_ref[...], g_ref, w_ref, b_ref, sig_ref, eenew_ref)


_EDGE_SPEC = pl.BlockSpec((_BE, _H), lambda i: (i, 0))
_EDGE_SHAPE = jax.ShapeDtypeStruct((_E2, _H), jnp.float32)
_SIG_SPEC = pl.BlockSpec((_NSC, _BE, _HC), lambda i: (0, i, 0))
_SIG_SHAPE = jax.ShapeDtypeStruct((_NSC, _E2, _HC), jnp.float32)


def _edge_wspecs():
    return [_EDGE_SPEC,
            pl.BlockSpec((_H, _H), lambda i: (0, 0)),
            pl.BlockSpec((1, _H), lambda i: (0, 0))]


def _tc_edge0(e, we, be, g, w4, b4, off):
    return pl.pallas_call(
        _edge0_body,
        grid=(_E2 // _BE,),
        in_specs=[pl.BlockSpec((_BE, 1), lambda i: (i + off, 0)),
                  pl.BlockSpec((1, _H), lambda i: (0, 0)),
                  pl.BlockSpec((1, _H), lambda i: (0, 0))] + _edge_wspecs(),
        out_specs=[_SIG_SPEC, _EDGE_SPEC],
        out_shape=[_SIG_SHAPE, _EDGE_SHAPE],
    )(e, we, be, g, w4, b4)


def _tc_edge(ee, g, w4, b4, want_ee):
    out_specs = [_SIG_SPEC, _EDGE_SPEC] if want_ee else [_SIG_SPEC]
    out_shape = [_SIG_SHAPE, _EDGE_SHAPE] if want_ee else [_SIG_SHAPE]
    return pl.pallas_call(
        _edge_body,
        grid=(_E2 // _BE,),
        in_specs=[_EDGE_SPEC] + _edge_wspecs(),
        out_specs=out_specs,
        out_shape=out_shape,
    )(ee, g, w4, b4)


def _readout_body(hprev_ref, ahprev_ref, acc_ref,
                  w0, b0, w1, b1, w2, b2, y_ref):
    hh = _hupdate(hprev_ref, ahprev_ref, acc_ref)
    y = jnp.maximum(jnp.dot(hh, w0[...], preferred_element_type=jnp.float32)
                    + b0[...], 0.0)
    y = jnp.maximum(jnp.dot(y, w1[...], preferred_element_type=jnp.float32)
                    + b1[...], 0.0)
    y_ref[...] = jnp.dot(y, w2[...], preferred_element_type=jnp.float32) + b2[...]


def _tc_readout(hprev, ahprev, acc, W0, b0, W1, b1, W2, b2):
    H2, H4, NC = W0.shape[1], W1.shape[1], W2.shape[1]
    return pl.pallas_call(
        _readout_body,
        grid=(_N // _BN,),
        in_specs=[pl.BlockSpec((_BN, _H), lambda i: (i, 0)),
                  pl.BlockSpec((_BN, _H), lambda i: (i, 0))] + _ACC_SPECS +
                 [pl.BlockSpec((_H, H2), lambda i: (0, 0)),
                  pl.BlockSpec((1, H2), lambda i: (0, 0)),
                  pl.BlockSpec((H2, H4), lambda i: (0, 0)),
                  pl.BlockSpec((1, H4), lambda i: (0, 0)),
                  pl.BlockSpec((H4, NC), lambda i: (0, 0)),
                  pl.BlockSpec((1, NC), lambda i: (0, 0))],
        out_specs=pl.BlockSpec((_BN, NC), lambda i: (i, 0)),
        out_shape=jax.ShapeDtypeStruct((_N, NC), jnp.float32),
    )(hprev, ahprev, acc, W0, b0, W1, b1, W2, b2)


def kernel(h, edge_index, e, emb_h, We, be, layerW, layerB, W0, b0, W1, b1, W2, b2):
    L = layerW.shape[0]
    src = edge_index[0].astype(jnp.int32)
    dst = edge_index[1].astype(jnp.int32)
    srcH = src.reshape(2, _E2)
    dstH = dst.reshape(2, _E2)
    src_g = [srcH[p].reshape(_NSC, _NS, _NCHG, _RPC, _IR) for p in range(2)]
    dst_g = [dstH[p].reshape(_NSC, _NS, _NCHG, _RPC, _IR) for p in range(2)]
    src_s = [srcH[p].reshape(_NS, _NCHH, _RPCS, _IRS) for p in range(2)]
    dst_s = [dstH[p].reshape(_NS, _NCHH, _RPCS, _IRS) for p in range(2)]

    h2 = h.astype(jnp.int32).reshape(_N, 1)
    lW = layerW.astype(jnp.float32)
    lB = layerB.reshape(L, 5, 1, _H).astype(jnp.float32)

    hh = ah = acc = None
    eeh = [None, None]
    sigh = [None, None]
    for l in range(L):
        if l == 0:
            hh, ah, bh, dh, eh = _tc_node0(h2, emb_h, lW[0], lB[0])
        else:
            hh, ah, bh, dh, eh = _tc_node(hh, ah, acc, lW[l], lB[l])
        gh = [_sc_gather(dh, eh, src_g[p], dst_g[p]) for p in range(2)]
        w4 = lW[l, 4]
        b4 = lB[l, 4]
        for p in range(2):
            if l == 0:
                sigh[p], eeh[p] = _tc_edge0(
                    e, We.reshape(1, _H), be.reshape(1, _H),
                    gh[p], w4, b4, p * (_E2 // _BE))
            elif l < L - 1:
                sigh[p], eeh[p] = _tc_edge(eeh[p], gh[p], w4, b4, True)
            else:
                (sigh[p],) = _tc_edge(eeh[p], gh[p], w4, b4, False)
        acc = _sc_scatter(sigh[0], sigh[1], bh,
                          src_s[0], dst_s[0], src_s[1], dst_s[1])

    return _tc_readout(hh, ah, acc, W0, b0.reshape(1, -1), W1, b1.reshape(1, -1),
                       W2, b2.reshape(1, -1))
